# baseline (speedup 1.0000x reference)
import jax
import jax.numpy as jnp
from jax.experimental import pallas as pl
from jax.experimental.pallas import tpu as pltpu

_MARK_AX = 0
_OP_SHL = 1
_OP_SHR = 2
_ALU_LO = 3
_ALU_HI = 19
_AX_CARRY_LO = 35
_OUTPUT_LO = 51
_OUTPUT_HI = 67

_ROWS_PER_BLOCK = 2048


def _body(x_ref, o_ref):
    x = x_ref[...]
    r, f = x.shape
    lane = jax.lax.broadcasted_iota(jnp.int32, (r, f), 1)
    neg = jnp.float32(-jnp.inf)

    def window_argmax(lo):
        m = jnp.where((lane >= lo) & (lane < lo + 16), x, neg)
        return (jnp.argmax(m, axis=1).astype(jnp.int32) - lo)[:, None]

    val_lo = window_argmax(_ALU_LO)
    val_hi = window_argmax(_ALU_HI)
    shift_amt = jnp.minimum(window_argmax(_AX_CARRY_LO), 31)

    mark = x[:, _MARK_AX:_MARK_AX + 1] >= 0.5
    is_shl = x[:, _OP_SHL:_OP_SHL + 1] > 0.5
    is_shr = x[:, _OP_SHR:_OP_SHR + 1] > 0.5
    active = mark & (is_shl | is_shr)

    value = val_lo + (val_hi << 4)
    shl_res = (value << shift_amt) & 255
    shr_res = value >> shift_amt
    result = jnp.where(is_shl, shl_res, shr_res)

    res_lo = result & 15
    res_hi = result >> 4

    hit = (lane == res_lo + _OUTPUT_LO) | (lane == res_hi + _OUTPUT_HI)
    add = jnp.where(active & hit, jnp.float32(2.0), jnp.float32(0.0))
    o_ref[...] = x + add


def kernel(x_bd):
    b, s, f = x_bd.shape
    x2 = x_bd.reshape(b * s, f)
    n_rows = b * s
    out = pl.pallas_call(
        _body,
        grid=(n_rows // _ROWS_PER_BLOCK,),
        in_specs=[pl.BlockSpec((_ROWS_PER_BLOCK, f), lambda i: (i, 0))],
        out_specs=pl.BlockSpec((_ROWS_PER_BLOCK, f), lambda i: (i, 0)),
        out_shape=jax.ShapeDtypeStruct((n_rows, f), x_bd.dtype),
    )(x2)
    return out.reshape(b, s, f)
